# baseline (speedup 1.0000x reference)
import functools

import jax
import jax.numpy as jnp
from jax import lax
from jax.experimental import pallas as pl
from jax.experimental.pallas import tpu as pltpu
from jax.experimental.pallas import tpu_sc as plsc

B = 16384
D = 16
NC = 2
NS = 16
NW = NC * NS
BPW = B // NW
L = 16

_mesh = plsc.VectorSubcoreMesh(core_axis_name="c", subcore_axis_name="s")


@functools.partial(
    pl.kernel,
    mesh=_mesh,
    out_type=[
        jax.ShapeDtypeStruct((B, D), jnp.float32),
        jax.ShapeDtypeStruct((B, D), jnp.float32),
        jax.ShapeDtypeStruct((B, D), jnp.float32),
    ],
    scratch_types=[
        pltpu.VMEM((BPW,), jnp.int32),
        pltpu.VMEM((8 + BPW,), jnp.int32),
        pltpu.VMEM((BPW,), jnp.int32),
        pltpu.VMEM((BPW, D), jnp.float32),
        pltpu.VMEM((BPW, D), jnp.float32),
        pltpu.VMEM((BPW, D), jnp.float32),
        pltpu.SemaphoreType.DMA,
        pltpu.SemaphoreType.DMA,
        pltpu.SemaphoreType.DMA,
    ],
    compiler_params=pltpu.CompilerParams(use_tc_tiling_on_sc=False),
)
def _emb_lookup(idx1_hbm, idx2_hbm, table_hbm, out1, out2, out3,
                idx1_v, idx2x_v, idx3_v, rows1_v, rows2_v, rows3_v,
                sem1, sem2, sem3):
    wid = lax.axis_index("s") * NC + lax.axis_index("c")
    base = pl.multiple_of(wid * BPW, BPW)
    pltpu.sync_copy(idx1_hbm.at[pl.ds(base, BPW)], idx1_v)
    pad_off = pl.multiple_of(lax.rem(base + (B - 8), B), 8)
    pltpu.sync_copy(idx2_hbm.at[pl.ds(pad_off, 8)], idx2x_v.at[pl.ds(0, 8)])
    pltpu.sync_copy(idx2_hbm.at[pl.ds(base, BPW)], idx2x_v.at[pl.ds(8, BPW)])
    for j in range(BPW // L):
        idx3_v[pl.ds(j * L, L)] = idx2x_v[pl.ds(j * L + 7, L)]

    g1 = pltpu.async_copy(table_hbm.at[idx1_v], rows1_v, sem1)
    g2 = pltpu.async_copy(table_hbm.at[idx2x_v.at[pl.ds(8, BPW)]], rows2_v, sem2)
    g3 = pltpu.async_copy(table_hbm.at[idx3_v], rows3_v, sem3)
    g1.wait()
    pltpu.sync_copy(rows1_v, out1.at[pl.ds(base, BPW)])
    g2.wait()
    pltpu.sync_copy(rows2_v, out2.at[pl.ds(base, BPW)])
    g3.wait()
    pltpu.sync_copy(rows3_v, out3.at[pl.ds(base, BPW)])


def kernel(input_ids, node_embedding):
    idx1 = input_ids[:, 0]
    idx2 = input_ids[:, 1]
    out1, out2, out3 = _emb_lookup(idx1, idx2, node_embedding)
    return out1, out2, out3


# speedup vs baseline: 1.0030x; 1.0030x over previous
import functools

import jax
import jax.numpy as jnp
from jax import lax
from jax.experimental import pallas as pl
from jax.experimental.pallas import tpu as pltpu
from jax.experimental.pallas import tpu_sc as plsc

B = 16384
D = 16
NC = 2
NS = 16
NW = NC * NS
BPW = B // NW

_mesh = plsc.VectorSubcoreMesh(core_axis_name="c", subcore_axis_name="s")


@functools.partial(
    pl.kernel,
    mesh=_mesh,
    out_type=[
        jax.ShapeDtypeStruct((B, D), jnp.float32),
        jax.ShapeDtypeStruct((B, D), jnp.float32),
        jax.ShapeDtypeStruct((B, D), jnp.float32),
    ],
    scratch_types=[
        pltpu.VMEM((BPW,), jnp.int32),
        pltpu.VMEM((BPW,), jnp.int32),
        pltpu.VMEM((BPW, D), jnp.float32),
        pltpu.VMEM((BPW, D), jnp.float32),
        pltpu.SemaphoreType.DMA,
        pltpu.SemaphoreType.DMA,
        pltpu.SemaphoreType.DMA,
    ],
    compiler_params=pltpu.CompilerParams(use_tc_tiling_on_sc=False),
)
def _emb_lookup(idx1_hbm, idx2_hbm, table_hbm, out1, out2, out3,
                idx1_v, idx2_v, rows1_v, rows2_v,
                sem1, sem2, semo):
    wid = lax.axis_index("s") * NC + lax.axis_index("c")
    base = pl.multiple_of(wid * BPW, BPW)
    pltpu.sync_copy(idx1_hbm.at[pl.ds(base, BPW)], idx1_v)
    pltpu.sync_copy(idx2_hbm.at[pl.ds(base, BPW)], idx2_v)

    g1 = pltpu.async_copy(table_hbm.at[idx1_v], rows1_v, sem1)
    g2 = pltpu.async_copy(table_hbm.at[idx2_v], rows2_v, sem2)
    g1.wait()
    o1 = pltpu.async_copy(rows1_v, out1.at[pl.ds(base, BPW)], semo)
    g2.wait()
    o2 = pltpu.async_copy(rows2_v, out2.at[pl.ds(base, BPW)], semo)
    o3 = pltpu.async_copy(rows2_v.at[pl.ds(0, BPW - 1)],
                          out3.at[pl.ds(base + 1, BPW - 1)], semo)
    nxt = lax.rem(base + BPW, B)
    o4 = pltpu.async_copy(rows2_v.at[pl.ds(BPW - 1, 1)],
                          out3.at[pl.ds(nxt, 1)], semo)
    o1.wait()
    o2.wait()
    o3.wait()
    o4.wait()


def kernel(input_ids, node_embedding):
    idx1 = input_ids[:, 0]
    idx2 = input_ids[:, 1]
    out1, out2, out3 = _emb_lookup(idx1, idx2, node_embedding)
    return out1, out2, out3


# speedup vs baseline: 1.0049x; 1.0019x over previous
import functools

import jax
import jax.numpy as jnp
from jax import lax
from jax.experimental import pallas as pl
from jax.experimental.pallas import tpu as pltpu
from jax.experimental.pallas import tpu_sc as plsc

B = 16384
D = 16
NC = 2
NS = 16
NW = NC * NS
BPW = B // NW
L = 16

_mesh = plsc.VectorSubcoreMesh(core_axis_name="c", subcore_axis_name="s")


@functools.partial(
    pl.kernel,
    mesh=_mesh,
    out_type=[
        jax.ShapeDtypeStruct((B, D), jnp.float32),
        jax.ShapeDtypeStruct((B, D), jnp.float32),
        jax.ShapeDtypeStruct((B, D), jnp.float32),
    ],
    scratch_types=[
        pltpu.VMEM((2 * BPW,), jnp.int32),
        pltpu.VMEM((BPW,), jnp.int32),
        pltpu.VMEM((BPW,), jnp.int32),
        pltpu.VMEM((BPW, D), jnp.float32),
        pltpu.VMEM((BPW, D), jnp.float32),
        pltpu.SemaphoreType.DMA,
        pltpu.SemaphoreType.DMA,
        pltpu.SemaphoreType.DMA,
    ],
    compiler_params=pltpu.CompilerParams(use_tc_tiling_on_sc=False),
)
def _emb_lookup(ids_hbm, table_hbm, out1, out2, out3,
                pairs_v, idx1_v, idx2_v, rows1_v, rows2_v,
                sem1, sem2, semo):
    wid = lax.axis_index("s") * NC + lax.axis_index("c")
    base = pl.multiple_of(wid * BPW, BPW)
    pltpu.sync_copy(ids_hbm.at[pl.ds(2 * base, 2 * BPW)], pairs_v)
    lane = lax.iota(jnp.int32, L)
    ev_pat = (2 * lane) & (L - 1)
    od_pat = (2 * lane + 1) & (L - 1)
    lo_half = lane < (L // 2)
    dnums = lax.GatherDimensionNumbers(
        offset_dims=(), collapsed_slice_dims=(0,), start_index_map=(0,))

    def _vgather(vec, pat):
        return lax.gather(vec, pat[:, None], dnums, slice_sizes=(1,),
                          mode=lax.GatherScatterMode.PROMISE_IN_BOUNDS)

    for j in range(BPW // L):
        a = pairs_v[pl.ds(2 * j * L, L)]
        b = pairs_v[pl.ds(2 * j * L + L, L)]
        idx1_v[pl.ds(j * L, L)] = jnp.where(
            lo_half, _vgather(a, ev_pat), _vgather(b, ev_pat))
        idx2_v[pl.ds(j * L, L)] = jnp.where(
            lo_half, _vgather(a, od_pat), _vgather(b, od_pat))

    g1 = pltpu.async_copy(table_hbm.at[idx1_v], rows1_v, sem1)
    g2 = pltpu.async_copy(table_hbm.at[idx2_v], rows2_v, sem2)
    g1.wait()
    o1 = pltpu.async_copy(rows1_v, out1.at[pl.ds(base, BPW)], semo)
    g2.wait()
    o2 = pltpu.async_copy(rows2_v, out2.at[pl.ds(base, BPW)], semo)
    o3 = pltpu.async_copy(rows2_v.at[pl.ds(0, BPW - 1)],
                          out3.at[pl.ds(base + 1, BPW - 1)], semo)
    nxt = lax.rem(base + BPW, B)
    o4 = pltpu.async_copy(rows2_v.at[pl.ds(BPW - 1, 1)],
                          out3.at[pl.ds(nxt, 1)], semo)
    o1.wait()
    o2.wait()
    o3.wait()
    o4.wait()


def kernel(input_ids, node_embedding):
    ids_flat = input_ids.reshape(2 * B)
    out1, out2, out3 = _emb_lookup(ids_flat, node_embedding)
    return out1, out2, out3
